# baseline (speedup 1.0000x reference)
import functools
import jax
import jax.numpy as jnp
from jax import lax
from jax.experimental import pallas as pl
from jax.experimental.pallas import tpu as pltpu
from jax.experimental.pallas import tpu_sc as plsc

N_NODES = 10000
N_EDGES = 320000
D = 128
EPS = 1e-5

_NC = 2
_NS = 16
_L = 16
_NW = _NC * _NS
_EPW = N_EDGES // _NW
_NPAD = 10240
_STRIP = _NPAD // _NS

_sc_mesh = plsc.VectorSubcoreMesh(core_axis_name="c", subcore_axis_name="s",
                                  num_cores=_NC, num_subcores=_NS)


@functools.partial(
    pl.kernel,
    out_type=[jax.ShapeDtypeStruct((_NC, _NPAD), jnp.float32),
              jax.ShapeDtypeStruct((_NC, _NPAD), jnp.float32)],
    mesh=_sc_mesh,
    compiler_params=pltpu.CompilerParams(needs_layout_passes=False),
    scratch_types=[
        pltpu.VMEM((_EPW,), jnp.int32),
        pltpu.VMEM((_NPAD,), jnp.float32),
        pltpu.VMEM((_STRIP,), jnp.float32),
        pltpu.VMEM((_NS, _STRIP), jnp.float32),
        pltpu.VMEM_SHARED((_NS, _NPAD), jnp.float32),
    ],
)
def _hist_sc(src_hbm, nbr_hbm, deg_out, cnt_out,
             ids_v, hist_v, acc_v, tmp_v, shared):
    c = lax.axis_index("c")
    s = lax.axis_index("s")
    gw = c * _NS + s
    zeros = jnp.zeros((_L,), jnp.float32)
    ones = jnp.ones((_L,), jnp.float32)

    for in_hbm, out_hbm in ((src_hbm, deg_out), (nbr_hbm, cnt_out)):
        def zero_body(i, _):
            hist_v[pl.ds(i * _L, _L)] = zeros
            return 0
        lax.fori_loop(0, _NPAD // _L, zero_body, 0)
        pltpu.sync_copy(in_hbm.at[gw], ids_v)

        def hist_body(i, _):
            idx = ids_v[pl.ds(i * _L, _L)]
            plsc.addupdate_scatter(hist_v, [idx], ones)
            return 0
        lax.fori_loop(0, _EPW // _L, hist_body, 0)

        pltpu.sync_copy(hist_v, shared.at[s])
        plsc.subcore_barrier()

        pltpu.sync_copy(shared.at[:, pl.ds(s * _STRIP, _STRIP)], tmp_v)

        def red_body(i, _):
            d = pl.ds(i * _L, _L)
            v = tmp_v[0, d]
            for p in range(1, _NS):
                v = v + tmp_v[p, d]
            acc_v[d] = v
            return 0
        lax.fori_loop(0, _STRIP // _L, red_body, 0, unroll=2)

        pltpu.sync_copy(acc_v, out_hbm.at[c, pl.ds(s * _STRIP, _STRIP)])
        plsc.subcore_barrier()


def _gelu(h):
    return 0.5 * h * (1.0 + lax.erf(h * (2.0 ** -0.5)))


_K = 40
_NCH = _EPW // _K

import numpy as _np_perm
_PERM = _np_perm.empty((D,), _np_perm.int32)
for _j in range(D // 32):
    for _t in range(16):
        _PERM[32 * _j + 2 * _t] = 32 * _j + _t
        _PERM[32 * _j + 2 * _t + 1] = 32 * _j + 16 + _t


_GDN = lax.GatherDimensionNumbers(offset_dims=(), collapsed_slice_dims=(0,),
                                  start_index_map=(0,))


def _lane_bcast(v16, lane):
    idx = jnp.full((_L,), lane, jnp.int32)
    return lax.gather(v16, idx[:, None], _GDN, slice_sizes=(1,),
                      mode=lax.GatherScatterMode.PROMISE_IN_BOUNDS)


_NGB = 3
_NWB = 2
_NIB = 6
_UNR = 6


@functools.partial(
    pl.kernel,
    out_type=jax.ShapeDtypeStruct((_NC, _NPAD, D), jnp.float32),
    mesh=_sc_mesh,
    compiler_params=pltpu.CompilerParams(needs_layout_passes=False,
                                         use_tc_tiling_on_sc=False),
    scratch_types=(
        [pltpu.VMEM((3, _K), jnp.int32) for _ in range(_NIB)] +
        [pltpu.VMEM((_K, D // 2), jnp.int32) for _ in range(_NGB)] +
        [pltpu.VMEM((_K, D), jnp.float32) for _ in range(_NWB)] + [
            pltpu.VMEM_SHARED((_NPAD, D), jnp.float32),
            pltpu.SemaphoreType.DMA,
            pltpu.SemaphoreType.DMA,
            pltpu.SemaphoreType.DMA,
        ]),
)
def _spmm_sc(fx_hbm, idx_hbm, parts_out,
             ib0, ib1, ib2, ib3, ib4, ib5, gb0, gb1, gb2, wb0, wb1,
             acc, sem_g, sem_s, sem_i):
    idx_bufs = (ib0, ib1, ib2, ib3, ib4, ib5)
    gat_bufs = (gb0, gb1, gb2)
    wgt_bufs = (wb0, wb1)
    c = lax.axis_index("c")
    s = lax.axis_index("s")
    gw = c * _NS + s
    zeros = jnp.zeros((_L,), jnp.float32)
    last = _NCH - 1

    def zrow(e, _):
        for j in range(D // _L):
            wb0[e, pl.ds(j * _L, _L)] = zeros
        return 0
    lax.fori_loop(0, _K, zrow, 0)
    for b in range(_STRIP // _K):
        pltpu.sync_copy(wb0, acc.at[pl.ds(s * _STRIP + b * _K, _K)])
    plsc.subcore_barrier()

    def idx_start(k, b):
        pltpu.async_copy(idx_hbm.at[gw, k], idx_bufs[b], sem_i)

    def idx_wait(k, b):
        pltpu.make_async_copy(idx_hbm.at[gw, k], idx_bufs[b], sem_i).wait()

    def gather_start(g, b):
        pltpu.async_copy(fx_hbm.at[idx_bufs[b].at[0]], gat_bufs[g], sem_g)

    def gather_wait(g, b):
        pltpu.make_async_copy(fx_hbm.at[idx_bufs[b].at[0]], gat_bufs[g],
                              sem_g).wait()

    def scatter_start(w, b):
        pltpu.async_copy(wgt_bufs[w], acc.at[idx_bufs[b].at[1]], sem_s,
                         add=True)

    def scatter_wait(w, b):
        pltpu.make_async_copy(wgt_bufs[w], acc.at[idx_bufs[b].at[1]],
                              sem_s).wait()

    def weight_mul(g, w, b):
        gat = gat_bufs[g]
        out = wgt_bufs[w]
        wref = idx_bufs[b]
        mask = jnp.full((_L,), -65536, jnp.int32)

        def wm(e, _):
            wbits = wref[2, pl.ds(e & ~15, _L)]
            wl = _lane_bcast(plsc.bitcast(wbits, jnp.float32), e & 15)
            for j in range(D // 32):
                v = gat[e, pl.ds(j * _L, _L)]
                lo = plsc.bitcast(v << 16, jnp.float32)
                hi = plsc.bitcast(v & mask, jnp.float32)
                out[e, pl.ds(j * 32, _L)] = lo * wl
                out[e, pl.ds(j * 32 + _L, _L)] = hi * wl
            return 0
        lax.fori_loop(0, _K, wm, 0, unroll=4)

    def process(k, j):
        static = isinstance(k, int)
        gather_wait(j % _NGB, j % _NIB)
        if not static or k >= 2:
            scatter_wait((j - 2) % _NWB, (j - 2) % _NIB)
        if not static or k + 2 <= last:
            idx_wait(k + 2, (j + 2) % _NIB)
            gather_start((j + 2) % _NGB, (j + 2) % _NIB)
        if not static or k + 4 <= last:
            idx_start(k + 4, (j + 4) % _NIB)
        weight_mul(j % _NGB, j % _NWB, j % _NIB)
        scatter_start(j % _NWB, j % _NIB)

    for k in range(4):
        idx_start(k, k % _NIB)
    idx_wait(0, 0)
    gather_start(0, 0)
    idx_wait(1, 1)
    gather_start(1, 1)

    process(0, 0)
    process(1, 1)

    def body(i, _):
        k0 = 2 + i * _UNR
        for j in range(_UNR):
            process(k0 + j, 2 + j)
        return 0

    n_body = (_NCH - 2 - 8) // _UNR
    lax.fori_loop(0, n_body, body, 0)
    for k in range(2 + n_body * _UNR, _NCH):
        process(k, k)
    scatter_wait((last - 1) % _NWB, (last - 1) % _NIB)
    scatter_wait(last % _NWB, last % _NIB)

    plsc.subcore_barrier()
    pltpu.sync_copy(acc.at[pl.ds(s * _STRIP, _STRIP)],
                    parts_out.at[c, pl.ds(s * _STRIP, _STRIP)])


def _pre_ffn_body(x_ref, s_ref, g1_ref, b1_ref, w1_ref, l1_ref,
                  g2_ref, b2_ref, w2_ref, l2_ref, out_ref):
    x = x_ref[...]
    s = s_ref[...]
    mu = jnp.dot(s, x, preferred_element_type=jnp.float32)
    msq = jnp.dot(s, x * x, preferred_element_type=jnp.float32)
    var = msq - mu * mu
    xn = g1_ref[...] * (x - mu) * lax.rsqrt(var + EPS) + b1_ref[...]
    z = _gelu(jnp.dot(xn, w1_ref[...], preferred_element_type=jnp.float32)
              + l1_ref[...])
    mu2 = jnp.dot(s, z, preferred_element_type=jnp.float32)
    msq2 = jnp.dot(s, z * z, preferred_element_type=jnp.float32)
    var2 = msq2 - mu2 * mu2
    zn = g2_ref[...] * (z - mu2) * lax.rsqrt(var2 + EPS) + b2_ref[...]
    out_ref[...] = _gelu(
        jnp.dot(zn, w2_ref[...], preferred_element_type=jnp.float32)
        + l2_ref[...])


def _upd_ffn_body(x_ref, parts_ref, deg_ref,
                  g1x_ref, b1x_ref, g1a_ref, b1a_ref,
                  w1x_ref, w1a_ref, l1_ref,
                  g2_ref, b2_ref, w2_ref, l2_ref, out_ref):
    x = x_ref[...]
    deg = deg_ref[...]
    scale = jnp.where(deg > 0, 1.0 / (deg * float(D)), 0.0)
    p = parts_ref[...]
    agg = (p[0, :N_NODES] + p[1, :N_NODES]) * scale
    n = float(N_NODES)
    mux = jnp.mean(x, axis=0, keepdims=True)
    varx = jnp.mean(x * x, axis=0, keepdims=True) - mux * mux
    mua = jnp.mean(agg, axis=0, keepdims=True)
    vara = jnp.mean(agg * agg, axis=0, keepdims=True) - mua * mua
    xn = g1x_ref[...] * (x - mux) * lax.rsqrt(varx + EPS) + b1x_ref[...]
    an = g1a_ref[...] * (agg - mua) * lax.rsqrt(vara + EPS) + b1a_ref[...]
    h = _gelu(jnp.dot(xn, w1x_ref[...], preferred_element_type=jnp.float32)
              + jnp.dot(an, w1a_ref[...], preferred_element_type=jnp.float32)
              + l1_ref[...])
    muh = jnp.mean(h, axis=0, keepdims=True)
    varh = jnp.mean(h * h, axis=0, keepdims=True) - muh * muh
    hn = g2_ref[...] * (h - muh) * lax.rsqrt(varh + EPS) + b2_ref[...]
    out_ref[...] = _gelu(
        jnp.dot(hn, w2_ref[...], preferred_element_type=jnp.float32)
        + l2_ref[...])


def _vmem_call(body, out_shape, n_in):
    return pl.pallas_call(
        body,
        out_shape=out_shape,
        in_specs=[pl.BlockSpec(memory_space=pltpu.VMEM)] * n_in,
        out_specs=pl.BlockSpec(memory_space=pltpu.VMEM),
    )


def kernel(node_representations, edges, edge_weights,
           pre_bn1_g, pre_bn1_b, pre_w1, pre_b1,
           pre_bn2_g, pre_bn2_b, pre_w2, pre_b2,
           upd_bn1_g, upd_bn1_b, upd_w1, upd_b1,
           upd_bn2_g, upd_bn2_b, upd_w2, upd_b2):
    x = node_representations
    src = edges[0]
    nbr = edges[1]

    src_w = src.reshape(_NW, _EPW)
    nbr_w = nbr.reshape(_NW, _EPW)
    deg2, cnt2 = _hist_sc(src_w, nbr_w)
    cnt = (cnt2[0] + cnt2[1])[:N_NODES]
    deg = (deg2[0] + deg2[1])[:N_NODES]

    s = (cnt * (1.0 / N_EDGES))[None, :]
    r2 = lambda v: v[None, :]
    fx = _vmem_call(_pre_ffn_body,
                    jax.ShapeDtypeStruct((N_NODES, D), jnp.float32), 10)(
        x, s, r2(pre_bn1_g), r2(pre_bn1_b), pre_w1, r2(pre_b1),
        r2(pre_bn2_g), r2(pre_bn2_b), pre_w2, r2(pre_b2))

    nbr_k = nbr.reshape(_NW, _NCH, _K)
    src_k = src.reshape(_NW, _NCH, _K)
    w_bits = lax.bitcast_convert_type(edge_weights,
                                      jnp.int32).reshape(_NW, _NCH, _K)
    idx_k = jnp.stack([nbr_k, src_k, w_bits], axis=2)
    fxp = jnp.take(fx.astype(jnp.bfloat16), jnp.asarray(_PERM), axis=1)
    fxp32 = lax.bitcast_convert_type(fxp.reshape(N_NODES, D // 2, 2),
                                     jnp.int32)
    parts = _spmm_sc(fxp32, idx_k)

    out = _vmem_call(_upd_ffn_body,
                     jax.ShapeDtypeStruct((N_NODES, D), jnp.float32), 14)(
        x, parts, deg[:, None],
        r2(upd_bn1_g[:D]), r2(upd_bn1_b[:D]),
        r2(upd_bn1_g[D:]), r2(upd_bn1_b[D:]),
        upd_w1[:D], upd_w1[D:], r2(upd_b1),
        r2(upd_bn2_g), r2(upd_bn2_b), upd_w2, r2(upd_b2))
    return out


# speedup vs baseline: 1.7002x; 1.7002x over previous
import functools
import jax
import jax.numpy as jnp
from jax import lax
from jax.experimental import pallas as pl
from jax.experimental.pallas import tpu as pltpu
from jax.experimental.pallas import tpu_sc as plsc

N_NODES = 10000
N_EDGES = 320000
D = 128
EPS = 1e-5

_NC = 2
_NS = 16
_L = 16
_NW = _NC * _NS
_EPW = N_EDGES // _NW
_NPAD = 10240
_STRIP = _NPAD // _NS

_sc_mesh = plsc.VectorSubcoreMesh(core_axis_name="c", subcore_axis_name="s",
                                  num_cores=_NC, num_subcores=_NS)


@functools.partial(
    pl.kernel,
    out_type=[jax.ShapeDtypeStruct((_NC, _NPAD), jnp.float32),
              jax.ShapeDtypeStruct((_NC, _NPAD), jnp.float32)],
    mesh=_sc_mesh,
    compiler_params=pltpu.CompilerParams(needs_layout_passes=False),
    scratch_types=[
        pltpu.VMEM((_EPW,), jnp.int32),
        pltpu.VMEM((_NPAD,), jnp.float32),
        pltpu.VMEM((_STRIP,), jnp.float32),
        pltpu.VMEM((_NS, _STRIP), jnp.float32),
        pltpu.VMEM_SHARED((_NS, _NPAD), jnp.float32),
    ],
)
def _hist_sc(src_hbm, nbr_hbm, deg_out, cnt_out,
             ids_v, hist_v, acc_v, tmp_v, shared):
    c = lax.axis_index("c")
    s = lax.axis_index("s")
    gw = c * _NS + s
    zeros = jnp.zeros((_L,), jnp.float32)
    ones = jnp.ones((_L,), jnp.float32)

    for in_hbm, out_hbm in ((src_hbm, deg_out), (nbr_hbm, cnt_out)):
        def zero_body(i, _):
            hist_v[pl.ds(i * _L, _L)] = zeros
            return 0
        lax.fori_loop(0, _NPAD // _L, zero_body, 0)
        pltpu.sync_copy(in_hbm.at[gw], ids_v)

        def hist_body(i, _):
            idx = ids_v[pl.ds(i * _L, _L)]
            plsc.addupdate_scatter(hist_v, [idx], ones)
            return 0
        lax.fori_loop(0, _EPW // _L, hist_body, 0)

        pltpu.sync_copy(hist_v, shared.at[s])
        plsc.subcore_barrier()

        pltpu.sync_copy(shared.at[:, pl.ds(s * _STRIP, _STRIP)], tmp_v)

        def red_body(i, _):
            d = pl.ds(i * _L, _L)
            v = tmp_v[0, d]
            for p in range(1, _NS):
                v = v + tmp_v[p, d]
            acc_v[d] = v
            return 0
        lax.fori_loop(0, _STRIP // _L, red_body, 0, unroll=2)

        pltpu.sync_copy(acc_v, out_hbm.at[c, pl.ds(s * _STRIP, _STRIP)])
        plsc.subcore_barrier()


def _gelu(h):
    return 0.5 * h * (1.0 + lax.erf(h * (2.0 ** -0.5)))


_K = 80
_NCH = _EPW // _K

_GDN = lax.GatherDimensionNumbers(offset_dims=(), collapsed_slice_dims=(0,),
                                  start_index_map=(0,))


def _lane_bcast(v16, lane):
    idx = jnp.full((_L,), lane, jnp.int32)
    return lax.gather(v16, idx[:, None], _GDN, slice_sizes=(1,),
                      mode=lax.GatherScatterMode.PROMISE_IN_BOUNDS)


_NRB = 3
_NIB = 5
_UNR = 15


@functools.partial(
    pl.kernel,
    out_type=jax.ShapeDtypeStruct((_NC, _NPAD, D), jnp.float32),
    mesh=_sc_mesh,
    compiler_params=pltpu.CompilerParams(needs_layout_passes=False),
    scratch_types=(
        [pltpu.VMEM((3, _K), jnp.int32) for _ in range(_NIB)] +
        [pltpu.VMEM((_K, D), jnp.float32) for _ in range(_NRB)] + [
            pltpu.VMEM_SHARED((_NPAD, D), jnp.float32),
            pltpu.SemaphoreType.DMA,
            pltpu.SemaphoreType.DMA,
            pltpu.SemaphoreType.DMA,
        ]),
)
def _spmm_sc(fx_hbm, idx_hbm, parts_out,
             ib0, ib1, ib2, ib3, ib4, rb0, rb1, rb2,
             acc, sem_g, sem_s, sem_i):
    idx_bufs = (ib0, ib1, ib2, ib3, ib4)
    rows_bufs = (rb0, rb1, rb2)
    c = lax.axis_index("c")
    s = lax.axis_index("s")
    gw = c * _NS + s
    zeros = jnp.zeros((_L,), jnp.float32)
    last = _NCH - 1

    def zrow(e, _):
        for j in range(D // _L):
            rb0[e, pl.ds(j * _L, _L)] = zeros
        return 0
    lax.fori_loop(0, _K, zrow, 0)
    for b in range(_STRIP // _K):
        pltpu.sync_copy(rb0, acc.at[pl.ds(s * _STRIP + b * _K, _K)])
    plsc.subcore_barrier()

    def idx_start(k, b):
        pltpu.async_copy(idx_hbm.at[gw, k], idx_bufs[b], sem_i)

    def idx_wait(k, b):
        pltpu.make_async_copy(idx_hbm.at[gw, k], idx_bufs[b], sem_i).wait()

    def gather_start(r, b):
        pltpu.async_copy(fx_hbm.at[idx_bufs[b].at[0]], rows_bufs[r], sem_g)

    def gather_wait(r, b):
        pltpu.make_async_copy(fx_hbm.at[idx_bufs[b].at[0]], rows_bufs[r],
                              sem_g).wait()

    def scatter_start(r, b):
        pltpu.async_copy(rows_bufs[r], acc.at[idx_bufs[b].at[1]], sem_s,
                         add=True)

    def scatter_wait(r, b):
        pltpu.make_async_copy(rows_bufs[r], acc.at[idx_bufs[b].at[1]],
                              sem_s).wait()

    def weight_mul(r, b):
        rows = rows_bufs[r]
        wref = idx_bufs[b]

        def wm(e, _):
            wbits = wref[2, pl.ds(e & ~15, _L)]
            wl = _lane_bcast(plsc.bitcast(wbits, jnp.float32), e & 15)
            for j in range(D // _L):
                d = pl.ds(j * _L, _L)
                rows[e, d] = rows[e, d] * wl
            return 0
        lax.fori_loop(0, _K, wm, 0, unroll=8)

    def process(k, j):
        static = isinstance(k, int)
        gather_wait(j % _NRB, j % _NIB)
        weight_mul(j % _NRB, j % _NIB)
        scatter_start(j % _NRB, j % _NIB)
        if not static or k >= 1:
            scatter_wait((j - 1) % _NRB, (j - 1) % _NIB)
        if not static or k + 2 <= last:
            idx_wait(k + 2, (j + 2) % _NIB)
            gather_start((j + 2) % _NRB, (j + 2) % _NIB)
        if not static or k + 4 <= last:
            idx_start(k + 4, (j + 4) % _NIB)

    for k in range(4):
        idx_start(k, k % _NIB)
    idx_wait(0, 0)
    gather_start(0, 0)
    idx_wait(1, 1)
    gather_start(1, 1)

    process(0, 0)
    process(1, 1)

    def body(i, _):
        k0 = 2 + i * _UNR
        for j in range(_UNR):
            process(k0 + j, 2 + j)
        return 0

    n_body = (_NCH - 5) // _UNR
    lax.fori_loop(0, n_body, body, 0)
    for k in range(2 + n_body * _UNR, _NCH):
        process(k, k)
    scatter_wait(last % _NRB, last % _NIB)
    idx_wait(_NCH, _NCH % _NIB)

    plsc.subcore_barrier()
    pltpu.sync_copy(acc.at[pl.ds(s * _STRIP, _STRIP)],
                    parts_out.at[c, pl.ds(s * _STRIP, _STRIP)])


def _pre_ffn_body(x_ref, s_ref, g1_ref, b1_ref, w1_ref, l1_ref,
                  g2_ref, b2_ref, w2_ref, l2_ref, out_ref):
    x = x_ref[...]
    s = s_ref[...]
    mu = jnp.dot(s, x, preferred_element_type=jnp.float32)
    msq = jnp.dot(s, x * x, preferred_element_type=jnp.float32)
    var = msq - mu * mu
    xn = g1_ref[...] * (x - mu) * lax.rsqrt(var + EPS) + b1_ref[...]
    z = _gelu(jnp.dot(xn, w1_ref[...], preferred_element_type=jnp.float32)
              + l1_ref[...])
    mu2 = jnp.dot(s, z, preferred_element_type=jnp.float32)
    msq2 = jnp.dot(s, z * z, preferred_element_type=jnp.float32)
    var2 = msq2 - mu2 * mu2
    zn = g2_ref[...] * (z - mu2) * lax.rsqrt(var2 + EPS) + b2_ref[...]
    out_ref[...] = _gelu(
        jnp.dot(zn, w2_ref[...], preferred_element_type=jnp.float32)
        + l2_ref[...])


def _upd_ffn_body(x_ref, parts_ref, deg_ref,
                  g1x_ref, b1x_ref, g1a_ref, b1a_ref,
                  w1x_ref, w1a_ref, l1_ref,
                  g2_ref, b2_ref, w2_ref, l2_ref, out_ref):
    x = x_ref[...]
    deg = deg_ref[...]
    scale = jnp.where(deg > 0, 1.0 / (deg * float(D)), 0.0)
    p = parts_ref[...]
    agg = (p[0, :N_NODES] + p[1, :N_NODES]) * scale
    n = float(N_NODES)
    mux = jnp.mean(x, axis=0, keepdims=True)
    varx = jnp.mean(x * x, axis=0, keepdims=True) - mux * mux
    mua = jnp.mean(agg, axis=0, keepdims=True)
    vara = jnp.mean(agg * agg, axis=0, keepdims=True) - mua * mua
    xn = g1x_ref[...] * (x - mux) * lax.rsqrt(varx + EPS) + b1x_ref[...]
    an = g1a_ref[...] * (agg - mua) * lax.rsqrt(vara + EPS) + b1a_ref[...]
    h = _gelu(jnp.dot(xn, w1x_ref[...], preferred_element_type=jnp.float32)
              + jnp.dot(an, w1a_ref[...], preferred_element_type=jnp.float32)
              + l1_ref[...])
    muh = jnp.mean(h, axis=0, keepdims=True)
    varh = jnp.mean(h * h, axis=0, keepdims=True) - muh * muh
    hn = g2_ref[...] * (h - muh) * lax.rsqrt(varh + EPS) + b2_ref[...]
    out_ref[...] = _gelu(
        jnp.dot(hn, w2_ref[...], preferred_element_type=jnp.float32)
        + l2_ref[...])


def _vmem_call(body, out_shape, n_in):
    return pl.pallas_call(
        body,
        out_shape=out_shape,
        in_specs=[pl.BlockSpec(memory_space=pltpu.VMEM)] * n_in,
        out_specs=pl.BlockSpec(memory_space=pltpu.VMEM),
    )


def kernel(node_representations, edges, edge_weights,
           pre_bn1_g, pre_bn1_b, pre_w1, pre_b1,
           pre_bn2_g, pre_bn2_b, pre_w2, pre_b2,
           upd_bn1_g, upd_bn1_b, upd_w1, upd_b1,
           upd_bn2_g, upd_bn2_b, upd_w2, upd_b2):
    x = node_representations
    src = edges[0]
    nbr = edges[1]

    src_w = src.reshape(_NW, _EPW)
    nbr_w = nbr.reshape(_NW, _EPW)
    deg2, cnt2 = _hist_sc(src_w, nbr_w)
    cnt = (cnt2[0] + cnt2[1])[:N_NODES]
    deg = (deg2[0] + deg2[1])[:N_NODES]

    s = (cnt * (1.0 / N_EDGES))[None, :]
    r2 = lambda v: v[None, :]
    fx = _vmem_call(_pre_ffn_body,
                    jax.ShapeDtypeStruct((N_NODES, D), jnp.float32), 10)(
        x, s, r2(pre_bn1_g), r2(pre_bn1_b), pre_w1, r2(pre_b1),
        r2(pre_bn2_g), r2(pre_bn2_b), pre_w2, r2(pre_b2))

    nbr_k = nbr.reshape(_NW, _NCH, _K)
    src_k = src.reshape(_NW, _NCH, _K)
    w_bits = lax.bitcast_convert_type(edge_weights,
                                      jnp.int32).reshape(_NW, _NCH, _K)
    idx_k = jnp.stack([nbr_k, src_k, w_bits], axis=2)
    idx_k = jnp.concatenate(
        [idx_k, jnp.zeros((_NW, 1, 3, _K), jnp.int32)], axis=1)
    parts = _spmm_sc(fx, idx_k)

    out = _vmem_call(_upd_ffn_body,
                     jax.ShapeDtypeStruct((N_NODES, D), jnp.float32), 14)(
        x, parts, deg[:, None],
        r2(upd_bn1_g[:D]), r2(upd_bn1_b[:D]),
        r2(upd_bn1_g[D:]), r2(upd_bn1_b[D:]),
        upd_w1[:D], upd_w1[D:], r2(upd_b1),
        r2(upd_bn2_g), r2(upd_bn2_b), upd_w2, r2(upd_b2))
    return out
